# Initial kernel scaffold; baseline (speedup 1.0000x reference)
import jax
import jax.numpy as jnp
from jax.experimental import pallas as pl


def kernel(points_xyz, points_features, W0, b0, g0, beta0, W1, b1, g1, beta1, W2, b2, g2, beta2):
    raise NotImplementedError("write your pallas kernel here")


# speedup vs baseline: 9.1842x; 9.1842x over previous
import functools

import jax
import jax.numpy as jnp
from jax import lax
from jax.experimental import pallas as pl
from jax.experimental.pallas import tpu as pltpu
from jax.experimental.pallas import tpu_sc as plsc

_B, _N, _M, _K, _D = 4, 8192, 512, 32, 32
_R2 = 0.25
_EPS = 1e-5
_PW = 48
_NW = 32
_ROWS = _B * _M * _K
_RPW = _ROWS // _NW
_CH = 128
_NCH = _RPW // _CH
_MB = 128
_RB = 2048


def _fps_body(x_ref, y_ref, z_ref, f0_ref, sx_ref, sy_ref, sz_ref):
    x = x_ref[...]
    y = y_ref[...]
    z = z_ref[...]
    iota = lax.broadcasted_iota(jnp.int32, (_B, _N), 1)
    miota = lax.broadcasted_iota(jnp.int32, (_B, _M), 1)

    def body(i, carry):
        dist, far, bx, by, bz = carry
        mask = iota == far
        sx = jnp.sum(jnp.where(mask, x, 0.0), axis=1, keepdims=True)
        sy = jnp.sum(jnp.where(mask, y, 0.0), axis=1, keepdims=True)
        sz = jnp.sum(jnp.where(mask, z, 0.0), axis=1, keepdims=True)
        bx = jnp.where(miota == i, sx, bx)
        by = jnp.where(miota == i, sy, by)
        bz = jnp.where(miota == i, sz, bz)
        dx = x - sx
        dy = y - sy
        dz = z - sz
        d = (dx * dx + dy * dy) + dz * dz
        dist = jnp.where(d < dist, d, dist)
        mx = jnp.max(dist, axis=1, keepdims=True)
        far = jnp.min(jnp.where(dist == mx, iota, _N), axis=1, keepdims=True)
        return dist, far, bx, by, bz

    dist0 = jnp.full((_B, _N), 1e10, jnp.float32)
    far0 = f0_ref[...].astype(jnp.int32)
    zb = jnp.zeros((_B, _M), jnp.float32)
    _, _, bx, by, bz = lax.fori_loop(0, _M, body, (dist0, far0, zb, zb, zb))
    sx_ref[...] = bx
    sy_ref[...] = by
    sz_ref[...] = bz


def _fps(x, y, z, far0f):
    out = jax.ShapeDtypeStruct((_B, _M), jnp.float32)
    return pl.pallas_call(
        _fps_body,
        out_shape=(out, out, out),
    )(x, y, z, far0f)


def _radius_body(x_ref, y_ref, z_ref, c_ref, nn_ref):
    x = x_ref[0]
    y = y_ref[0]
    z = z_ref[0]
    c3 = c_ref[0]
    qx = c3[:, 0:1]
    qy = c3[:, 1:2]
    qz = c3[:, 2:3]
    nx = (x * x + y * y) + z * z
    nq = (qx * qx + qy * qy) + qz * qz
    bf = lambda v: v.astype(jnp.bfloat16).astype(jnp.float32)
    dot = (bf(qx) * bf(x) + bf(qy) * bf(y)) + bf(qz) * bf(z)
    dm = ((-2.0 * dot) + nx) + nq
    iota = lax.broadcasted_iota(jnp.int32, (_MB, _N), 1)
    kio = lax.broadcasted_iota(jnp.int32, (_MB, _K), 1)
    midx = jnp.where(dm > _R2, _N, iota)

    def sel(j, carry):
        midx, out = carry
        mn = jnp.min(midx, axis=1, keepdims=True)
        out = jnp.where(kio == j, mn, out)
        midx = jnp.where(midx == mn, _N, midx)
        return midx, out

    _, out = lax.fori_loop(0, _K, sel, (midx, jnp.zeros((_MB, _K), jnp.int32)))
    out = jnp.where(out == _N, out[:, 0:1], out)
    nn_ref[0] = out


def _radius_nn(x3, y3, z3, cent):
    row = pl.BlockSpec((1, 1, _N), lambda b, m: (b, 0, 0))
    return pl.pallas_call(
        _radius_body,
        grid=(_B, _M // _MB),
        in_specs=[row, row, row,
                  pl.BlockSpec((1, _MB, 3), lambda b, m: (b, m, 0))],
        out_specs=pl.BlockSpec((1, _MB, _K), lambda b, m: (b, m, 0)),
        out_shape=jax.ShapeDtypeStruct((_B, _M, _K), jnp.int32),
    )(x3, y3, z3, cent)


def _sc_gather(table, gidx3):
    mesh = plsc.VectorSubcoreMesh(
        core_axis_name="c", subcore_axis_name="s", num_cores=2, num_subcores=16)

    @functools.partial(
        pl.kernel,
        out_type=jax.ShapeDtypeStruct((_ROWS, _PW), jnp.float32),
        mesh=mesh,
        scratch_types=[
            pltpu.VMEM((_CH,), jnp.int32),
            pltpu.VMEM((_CH, _PW), jnp.float32),
            pltpu.SemaphoreType.DMA,
        ],
        compiler_params=pltpu.CompilerParams(use_tc_tiling_on_sc=False),
    )
    def k(tab_hbm, idx_hbm, out_hbm, idx_v, rows_v, sem):
        w = lax.axis_index("s") * 2 + lax.axis_index("c")

        def body(j, _):
            pltpu.sync_copy(idx_hbm.at[w, j], idx_v)
            pltpu.async_copy(tab_hbm.at[idx_v], rows_v, sem).wait()
            pltpu.sync_copy(rows_v, out_hbm.at[pl.ds(w * _RPW + j * _CH, _CH)])
            return 0

        lax.fori_loop(0, _NCH, body, 0)

    return k(table, gidx3)


def _l1_body(g_ref, c_ref, w_ref, wc_ref, b_ref, y_ref, st_ref):
    y = jnp.dot(g_ref[...], w_ref[...], preferred_element_type=jnp.float32)
    y = y - jnp.dot(c_ref[...], wc_ref[...], preferred_element_type=jnp.float32)
    y = y + b_ref[...]
    y_ref[...] = y

    @pl.when(pl.program_id(0) == 0)
    def _():
        st_ref[...] = jnp.zeros_like(st_ref)

    st_ref[0:1, :] += jnp.sum(y, axis=0, keepdims=True)
    st_ref[1:2, :] += jnp.sum(y * y, axis=0, keepdims=True)


def _layer1(g, cexp, w0pT, w0x4T, b0):
    co = w0pT.shape[1]
    return pl.pallas_call(
        _l1_body,
        grid=(_ROWS // _RB,),
        in_specs=[
            pl.BlockSpec((_RB, _PW), lambda r: (r, 0)),
            pl.BlockSpec((_RB, 8), lambda r: (r, 0)),
            pl.BlockSpec((_PW, co), lambda r: (0, 0)),
            pl.BlockSpec((8, co), lambda r: (0, 0)),
            pl.BlockSpec((1, co), lambda r: (0, 0)),
        ],
        out_specs=[
            pl.BlockSpec((_RB, co), lambda r: (r, 0)),
            pl.BlockSpec((8, co), lambda r: (0, 0)),
        ],
        out_shape=[
            jax.ShapeDtypeStruct((_ROWS, co), jnp.float32),
            jax.ShapeDtypeStruct((8, co), jnp.float32),
        ],
    )(g, cexp, w0pT, w0x4T, b0)


def _lmid_body(y_ref, sc_ref, sh_ref, w_ref, b_ref, z_ref, st_ref):
    xa = jax.nn.relu(y_ref[...] * sc_ref[...] + sh_ref[...])
    z = jnp.dot(xa, w_ref[...], preferred_element_type=jnp.float32) + b_ref[...]
    z_ref[...] = z

    @pl.when(pl.program_id(0) == 0)
    def _():
        st_ref[...] = jnp.zeros_like(st_ref)

    st_ref[0:1, :] += jnp.sum(z, axis=0, keepdims=True)
    st_ref[1:2, :] += jnp.sum(z * z, axis=0, keepdims=True)


def _layer_mid(y, scale, shift, wT, b):
    ci, co = wT.shape
    return pl.pallas_call(
        _lmid_body,
        grid=(_ROWS // _RB,),
        in_specs=[
            pl.BlockSpec((_RB, ci), lambda r: (r, 0)),
            pl.BlockSpec((1, ci), lambda r: (0, 0)),
            pl.BlockSpec((1, ci), lambda r: (0, 0)),
            pl.BlockSpec((ci, co), lambda r: (0, 0)),
            pl.BlockSpec((1, co), lambda r: (0, 0)),
        ],
        out_specs=[
            pl.BlockSpec((_RB, co), lambda r: (r, 0)),
            pl.BlockSpec((8, co), lambda r: (0, 0)),
        ],
        out_shape=[
            jax.ShapeDtypeStruct((_ROWS, co), jnp.float32),
            jax.ShapeDtypeStruct((8, co), jnp.float32),
        ],
    )(y, scale, shift, wT, b)


def _pool_body(y_ref, sc_ref, sh_ref, o_ref):
    y = y_ref[...]
    xa = jax.nn.relu(y * sc_ref[...] + sh_ref[...])
    o_ref[...] = jnp.max(xa, axis=1)


def _pool(y3, scale3, shift3):
    qb = 64
    nq = _B * _M
    return pl.pallas_call(
        _pool_body,
        grid=(nq // qb,),
        in_specs=[
            pl.BlockSpec((qb, _K, 64), lambda r: (r, 0, 0)),
            pl.BlockSpec((1, 1, 64), lambda r: (0, 0, 0)),
            pl.BlockSpec((1, 1, 64), lambda r: (0, 0, 0)),
        ],
        out_specs=pl.BlockSpec((qb, 64), lambda r: (r, 0)),
        out_shape=jax.ShapeDtypeStruct((nq, 64), jnp.float32),
    )(y3, scale3, shift3)


def _bn_coeffs(st, g, beta):
    n = float(_ROWS)
    mean = st[0] / n
    var = st[1] / n - mean * mean
    scale = g / jnp.sqrt(var + _EPS)
    shift = beta - mean * scale
    return scale[None, :], shift[None, :]


def kernel(points_xyz, points_features, W0, b0, g0, beta0, W1, b1, g1, beta1,
           W2, b2, g2, beta2):
    x = points_xyz[:, 0, :]
    y = points_xyz[:, 1, :]
    z = points_xyz[:, 2, :]

    far0 = jax.random.randint(jax.random.key(7), (_B,), 0, _N)
    far0f = far0.astype(jnp.float32)[:, None]

    sx, sy, sz = _fps(x, y, z, far0f)

    cent = jnp.stack([sx, sy, sz], axis=-1)
    nn_idx = _radius_nn(x[:, None, :], y[:, None, :], z[:, None, :], cent)

    table = jnp.concatenate([
        jnp.transpose(points_xyz, (0, 2, 1)),
        jnp.transpose(points_features, (0, 2, 1)),
        jnp.zeros((_B, _N, _PW - 3 - _D), jnp.float32),
    ], axis=-1).reshape(_B * _N, _PW)
    gidx = (jnp.arange(_B, dtype=jnp.int32)[:, None, None] * _N + nn_idx)
    gidx3 = gidx.reshape(_NW, _NCH, _CH)

    G = _sc_gather(table, gidx3)

    cexp = jnp.broadcast_to(cent[:, :, None, :], (_B, _M, _K, 3))
    cexp = jnp.concatenate(
        [cexp.reshape(_ROWS, 3), jnp.zeros((_ROWS, 5), jnp.float32)], axis=-1)

    w0pT = jnp.concatenate(
        [W0.T, jnp.zeros((_PW - 35, W0.shape[0]), jnp.float32)], axis=0)
    w0x4T = jnp.concatenate(
        [W0[:, :3].T, jnp.zeros((5, W0.shape[0]), jnp.float32)], axis=0)

    y1, st1 = _layer1(G, cexp, w0pT, w0x4T, b0[None, :])
    sc1, sh1 = _bn_coeffs(st1, g0, beta0)
    y2, st2 = _layer_mid(y1, sc1, sh1, W1.T, b1[None, :])
    sc2, sh2 = _bn_coeffs(st2, g1, beta1)
    y3, st3 = _layer_mid(y2, sc2, sh2, W2.T, b2[None, :])
    sc3, sh3 = _bn_coeffs(st3, g2, beta2)

    pooled = _pool(y3.reshape(_B * _M, _K, 64),
                   sc3[:, None, :], sh3[:, None, :])

    out_xyz = jnp.stack([sx, sy, sz], axis=1)
    out_x = jnp.transpose(pooled.reshape(_B, _M, 64), (0, 2, 1))
    return (out_xyz, out_x)


# speedup vs baseline: 9.7654x; 1.0633x over previous
import functools

import jax
import jax.numpy as jnp
from jax import lax
from jax.experimental import pallas as pl
from jax.experimental.pallas import tpu as pltpu
from jax.experimental.pallas import tpu_sc as plsc

_B, _N, _M, _K, _D = 4, 8192, 512, 32, 32
_R2 = 0.25
_EPS = 1e-5
_PW = 48
_NW = 32
_ROWS = _B * _M * _K
_RPW = _ROWS // _NW
_CH = 128
_NCH = _RPW // _CH
_MB = 128
_RB = 2048


_N8 = _N // 8


def _fps_body(x_ref, y_ref, z_ref, f0_ref, sx_ref, sy_ref, sz_ref):
    x = x_ref[...]
    y = y_ref[...]
    z = z_ref[...]
    iota = (lax.broadcasted_iota(jnp.int32, (_B, 8, _N8), 1) * _N8
            + lax.broadcasted_iota(jnp.int32, (_B, 8, _N8), 2))
    miota = lax.broadcasted_iota(jnp.int32, (_B, _M), 1)

    def body(i, carry):
        dist, far, bx, by, bz = carry
        mask = iota == far
        sx = jnp.sum(jnp.where(mask, x, 0.0), axis=(1, 2), keepdims=True)
        sy = jnp.sum(jnp.where(mask, y, 0.0), axis=(1, 2), keepdims=True)
        sz = jnp.sum(jnp.where(mask, z, 0.0), axis=(1, 2), keepdims=True)
        bx = jnp.where(miota == i, sx.reshape(_B, 1), bx)
        by = jnp.where(miota == i, sy.reshape(_B, 1), by)
        bz = jnp.where(miota == i, sz.reshape(_B, 1), bz)
        dx = x - sx
        dy = y - sy
        dz = z - sz
        d = (dx * dx + dy * dy) + dz * dz
        dist = jnp.where(d < dist, d, dist)
        mx = jnp.max(dist, axis=(1, 2), keepdims=True)
        far = jnp.min(jnp.where(dist == mx, iota, _N), axis=(1, 2),
                      keepdims=True)
        return dist, far, bx, by, bz

    dist0 = jnp.full((_B, 8, _N8), 1e10, jnp.float32)
    far0 = f0_ref[...].astype(jnp.int32).reshape(_B, 1, 1)
    zb = jnp.zeros((_B, _M), jnp.float32)
    _, _, bx, by, bz = lax.fori_loop(0, _M, body, (dist0, far0, zb, zb, zb))
    sx_ref[...] = bx
    sy_ref[...] = by
    sz_ref[...] = bz


def _fps(x, y, z, far0f):
    out = jax.ShapeDtypeStruct((_B, _M), jnp.float32)
    return pl.pallas_call(
        _fps_body,
        out_shape=(out, out, out),
    )(x.reshape(_B, 8, _N8), y.reshape(_B, 8, _N8), z.reshape(_B, 8, _N8),
      far0f)


def _radius_body(x_ref, y_ref, z_ref, c_ref, nn_ref):
    x = x_ref[0]
    y = y_ref[0]
    z = z_ref[0]
    c3 = c_ref[0]
    qx = c3[:, 0:1]
    qy = c3[:, 1:2]
    qz = c3[:, 2:3]
    nx = (x * x + y * y) + z * z
    nq = (qx * qx + qy * qy) + qz * qz
    bf = lambda v: v.astype(jnp.bfloat16).astype(jnp.float32)
    dot = (bf(qx) * bf(x) + bf(qy) * bf(y)) + bf(qz) * bf(z)
    dm = ((-2.0 * dot) + nx) + nq
    iota = lax.broadcasted_iota(jnp.int32, (_MB, _N), 1)
    kio = lax.broadcasted_iota(jnp.int32, (_MB, _K), 1)
    midx = jnp.where(dm > _R2, _N, iota)

    def sel(j, carry):
        midx, out = carry
        mn = jnp.min(midx, axis=1, keepdims=True)
        out = jnp.where(kio == j, mn, out)
        midx = jnp.where(midx == mn, _N, midx)
        return midx, out

    _, out = lax.fori_loop(0, _K, sel, (midx, jnp.zeros((_MB, _K), jnp.int32)))
    out = jnp.where(out == _N, out[:, 0:1], out)
    nn_ref[0] = jnp.minimum(out, _N - 1)


def _radius_nn(x3, y3, z3, cent):
    row = pl.BlockSpec((1, 1, _N), lambda b, m: (b, 0, 0))
    return pl.pallas_call(
        _radius_body,
        grid=(_B, _M // _MB),
        in_specs=[row, row, row,
                  pl.BlockSpec((1, _MB, 3), lambda b, m: (b, m, 0))],
        out_specs=pl.BlockSpec((1, _MB, _K), lambda b, m: (b, m, 0)),
        out_shape=jax.ShapeDtypeStruct((_B, _M, _K), jnp.int32),
    )(x3, y3, z3, cent)


def _sc_gather(table, gidx3):
    mesh = plsc.VectorSubcoreMesh(
        core_axis_name="c", subcore_axis_name="s", num_cores=2, num_subcores=16)

    @functools.partial(
        pl.kernel,
        out_type=jax.ShapeDtypeStruct((_ROWS, _PW), jnp.float32),
        mesh=mesh,
        scratch_types=[
            pltpu.VMEM((_CH,), jnp.int32),
            pltpu.VMEM((_CH, _PW), jnp.float32),
            pltpu.SemaphoreType.DMA,
        ],
        compiler_params=pltpu.CompilerParams(use_tc_tiling_on_sc=False),
    )
    def k(tab_hbm, idx_hbm, out_hbm, idx_v, rows_v, sem):
        w = lax.axis_index("s") * 2 + lax.axis_index("c")

        def body(j, _):
            pltpu.sync_copy(idx_hbm.at[w, j], idx_v)
            pltpu.async_copy(tab_hbm.at[idx_v], rows_v, sem).wait()
            pltpu.sync_copy(rows_v, out_hbm.at[pl.ds(w * _RPW + j * _CH, _CH)])
            return 0

        lax.fori_loop(0, _NCH, body, 0)

    return k(table, gidx3)


def _l1_body(g_ref, c_ref, w_ref, wc_ref, b_ref, y_ref, st_ref):
    y = jnp.dot(g_ref[...], w_ref[...], preferred_element_type=jnp.float32)
    y = y - jnp.dot(c_ref[...], wc_ref[...], preferred_element_type=jnp.float32)
    y = y + b_ref[...]
    y_ref[...] = y

    @pl.when(pl.program_id(0) == 0)
    def _():
        st_ref[...] = jnp.zeros_like(st_ref)

    st_ref[0:1, :] += jnp.sum(y, axis=0, keepdims=True)
    st_ref[1:2, :] += jnp.sum(y * y, axis=0, keepdims=True)


def _layer1(g, cexp, w0pT, w0x4T, b0):
    co = w0pT.shape[1]
    return pl.pallas_call(
        _l1_body,
        grid=(_ROWS // _RB,),
        in_specs=[
            pl.BlockSpec((_RB, _PW), lambda r: (r, 0)),
            pl.BlockSpec((_RB, 8), lambda r: (r, 0)),
            pl.BlockSpec((_PW, co), lambda r: (0, 0)),
            pl.BlockSpec((8, co), lambda r: (0, 0)),
            pl.BlockSpec((1, co), lambda r: (0, 0)),
        ],
        out_specs=[
            pl.BlockSpec((_RB, co), lambda r: (r, 0)),
            pl.BlockSpec((8, co), lambda r: (0, 0)),
        ],
        out_shape=[
            jax.ShapeDtypeStruct((_ROWS, co), jnp.float32),
            jax.ShapeDtypeStruct((8, co), jnp.float32),
        ],
    )(g, cexp, w0pT, w0x4T, b0)


def _lmid_body(y_ref, sc_ref, sh_ref, w_ref, b_ref, z_ref, st_ref):
    xa = jax.nn.relu(y_ref[...] * sc_ref[...] + sh_ref[...])
    z = jnp.dot(xa, w_ref[...], preferred_element_type=jnp.float32) + b_ref[...]
    z_ref[...] = z

    @pl.when(pl.program_id(0) == 0)
    def _():
        st_ref[...] = jnp.zeros_like(st_ref)

    st_ref[0:1, :] += jnp.sum(z, axis=0, keepdims=True)
    st_ref[1:2, :] += jnp.sum(z * z, axis=0, keepdims=True)


def _layer_mid(y, scale, shift, wT, b):
    ci, co = wT.shape
    return pl.pallas_call(
        _lmid_body,
        grid=(_ROWS // _RB,),
        in_specs=[
            pl.BlockSpec((_RB, ci), lambda r: (r, 0)),
            pl.BlockSpec((1, ci), lambda r: (0, 0)),
            pl.BlockSpec((1, ci), lambda r: (0, 0)),
            pl.BlockSpec((ci, co), lambda r: (0, 0)),
            pl.BlockSpec((1, co), lambda r: (0, 0)),
        ],
        out_specs=[
            pl.BlockSpec((_RB, co), lambda r: (r, 0)),
            pl.BlockSpec((8, co), lambda r: (0, 0)),
        ],
        out_shape=[
            jax.ShapeDtypeStruct((_ROWS, co), jnp.float32),
            jax.ShapeDtypeStruct((8, co), jnp.float32),
        ],
    )(y, scale, shift, wT, b)


def _pool_body(y_ref, sc_ref, sh_ref, o_ref):
    y = y_ref[...]
    xa = jax.nn.relu(y * sc_ref[...] + sh_ref[...])
    o_ref[...] = jnp.max(xa, axis=1)


def _pool(y3, scale3, shift3):
    qb = 64
    nq = _B * _M
    return pl.pallas_call(
        _pool_body,
        grid=(nq // qb,),
        in_specs=[
            pl.BlockSpec((qb, _K, 64), lambda r: (r, 0, 0)),
            pl.BlockSpec((1, 1, 64), lambda r: (0, 0, 0)),
            pl.BlockSpec((1, 1, 64), lambda r: (0, 0, 0)),
        ],
        out_specs=pl.BlockSpec((qb, 64), lambda r: (r, 0)),
        out_shape=jax.ShapeDtypeStruct((nq, 64), jnp.float32),
    )(y3, scale3, shift3)


def _bn_coeffs(st, g, beta):
    n = float(_ROWS)
    mean = st[0] / n
    var = st[1] / n - mean * mean
    scale = g / jnp.sqrt(var + _EPS)
    shift = beta - mean * scale
    return scale[None, :], shift[None, :]


def kernel(points_xyz, points_features, W0, b0, g0, beta0, W1, b1, g1, beta1,
           W2, b2, g2, beta2):
    x = points_xyz[:, 0, :]
    y = points_xyz[:, 1, :]
    z = points_xyz[:, 2, :]

    far0 = jax.random.randint(jax.random.key(7), (_B,), 0, _N)
    far0f = far0.astype(jnp.float32)[:, None]

    sx, sy, sz = _fps(x, y, z, far0f)

    cent = jnp.stack([sx, sy, sz], axis=-1)
    nn_idx = _radius_nn(x[:, None, :], y[:, None, :], z[:, None, :], cent)

    table = jnp.concatenate([
        jnp.transpose(points_xyz, (0, 2, 1)),
        jnp.transpose(points_features, (0, 2, 1)),
        jnp.zeros((_B, _N, _PW - 3 - _D), jnp.float32),
    ], axis=-1).reshape(_B * _N, _PW)
    gidx = (jnp.arange(_B, dtype=jnp.int32)[:, None, None] * _N + nn_idx)
    gidx3 = gidx.reshape(_NW, _NCH, _CH)

    G = _sc_gather(table, gidx3)

    cexp = jnp.broadcast_to(cent[:, :, None, :], (_B, _M, _K, 3))
    cexp = jnp.concatenate(
        [cexp.reshape(_ROWS, 3), jnp.zeros((_ROWS, 5), jnp.float32)], axis=-1)

    w0pT = jnp.concatenate(
        [W0.T, jnp.zeros((_PW - 35, W0.shape[0]), jnp.float32)], axis=0)
    w0x4T = jnp.concatenate(
        [W0[:, :3].T, jnp.zeros((5, W0.shape[0]), jnp.float32)], axis=0)

    y1, st1 = _layer1(G, cexp, w0pT, w0x4T, b0[None, :])
    sc1, sh1 = _bn_coeffs(st1, g0, beta0)
    y2, st2 = _layer_mid(y1, sc1, sh1, W1.T, b1[None, :])
    sc2, sh2 = _bn_coeffs(st2, g1, beta1)
    y3, st3 = _layer_mid(y2, sc2, sh2, W2.T, b2[None, :])
    sc3, sh3 = _bn_coeffs(st3, g2, beta2)

    pooled = _pool(y3.reshape(_B * _M, _K, 64),
                   sc3[:, None, :], sh3[:, None, :])

    out_xyz = jnp.stack([sx, sy, sz], axis=1)
    out_x = jnp.transpose(pooled.reshape(_B, _M, 64), (0, 2, 1))
    return (out_xyz, out_x)
